# baseline (speedup 1.0000x reference)
import functools

import jax
import jax.numpy as jnp
from jax import lax
from jax.experimental import pallas as pl
from jax.experimental.pallas import tpu as pltpu
from jax.experimental.pallas import tpu_sc as plsc

N = 10000
E = 320000
D = 128

NC, NS = 2, 16
NW = NC * NS
K = 128
NCHUNK = 80
EPW = K * NCHUNK
EPAD = NW * EPW
NP = 10112
RPS = NP // NS
PAD_ROW = N
ZROWS = 64

_f32 = jnp.float32


NBUF = 3
NG = -(-NCHUNK // NBUF)


def _zero_accum(buf, accum, s):
    z16 = jnp.zeros((16,), _f32)

    def _zb(i, _):
        for j in range(8):
            buf[i, pl.ds(j * 16, 16)] = z16
        return 0
    lax.fori_loop(0, K, _zb, 0)

    def _za(i, _):
        pltpu.sync_copy(buf, accum.at[pl.ds(s * RPS + i * K, K)])
        return 0
    lax.fori_loop(0, RPS // K, _za, 0)
    pltpu.sync_copy(buf, accum.at[pl.ds(s * RPS + RPS - K, K)])


def _sc_agg_body(h_hbm, srcf, dstf, agg0, agg1,
                 sidx, didx, rows0, rows1, rows2, accum, sem0, sem1, sem2):
    c = lax.axis_index("c")
    s = lax.axis_index("s")
    wid = s * NC + c
    base = wid * EPW
    rows = [rows0, rows1, rows2]
    sems = [sem0, sem1, sem2]

    _zero_accum(rows0, accum, s)
    plsc.subcore_barrier()

    def _ld(slot, j):
        off = pl.multiple_of(base + j * K, K)
        pltpu.sync_copy(srcf.at[pl.ds(off, K)], sidx.at[slot])
        pltpu.sync_copy(dstf.at[pl.ds(off, K)], didx.at[slot])

    def _gather(slot):
        pltpu.async_copy(h_hbm.at[sidx.at[slot]], rows[slot], sems[slot])

    def _gwait(slot):
        pltpu.make_async_copy(
            h_hbm.at[sidx.at[slot]], rows[slot], sems[slot]).wait()

    _ld(0, 0)
    _gather(0)
    _ld(1, 1)
    _gather(1)

    def _group(p, _):
        for b in range(NBUF):
            q = NBUF * p + b
            qn = q + 2
            nb = (b + 2) % NBUF

            @pl.when(qn < NCHUNK)
            def _():
                _ld(nb, qn)

            @pl.when(q < NCHUNK)
            def _():
                _gwait(b)

                @pl.when(qn < NCHUNK)
                def _():
                    _gather(nb)
                pltpu.sync_copy(rows[b], accum.at[didx.at[b]], add=True)
        return 0
    lax.fori_loop(0, NG, _group, 0)

    plsc.subcore_barrier()

    @pl.when(c == 0)
    def _():
        pltpu.sync_copy(accum.at[pl.ds(s * RPS, RPS)],
                        agg0.at[pl.ds(s * RPS, RPS)])

    @pl.when(c == 1)
    def _():
        pltpu.sync_copy(accum.at[pl.ds(s * RPS, RPS)],
                        agg1.at[pl.ds(s * RPS, RPS)])


def _sc_deg_body(dstf, deg0, deg1, didx, ones, zbuf, accum):
    c = lax.axis_index("c")
    s = lax.axis_index("s")
    wid = s * NC + c
    base = wid * EPW

    o16 = jnp.ones((16,), _f32)

    def _fill(i, _):
        for j in range(8):
            ones[i, pl.ds(j * 16, 16)] = o16
        return 0
    lax.fori_loop(0, K, _fill, 0)

    _zero_accum(zbuf, accum, s)
    plsc.subcore_barrier()

    def _chunk(j, _):
        off = pl.multiple_of(base + j * K, K)
        pltpu.sync_copy(dstf.at[pl.ds(off, K)], didx.at[0])
        pltpu.sync_copy(ones, accum.at[didx.at[0]], add=True)
        return 0
    lax.fori_loop(0, NCHUNK, _chunk, 0)

    plsc.subcore_barrier()

    @pl.when(c == 0)
    def _():
        pltpu.sync_copy(accum.at[pl.ds(s * RPS, RPS)],
                        deg0.at[pl.ds(s * RPS, RPS)])

    @pl.when(c == 1)
    def _():
        pltpu.sync_copy(accum.at[pl.ds(s * RPS, RPS)],
                        deg1.at[pl.ds(s * RPS, RPS)])


@functools.lru_cache(maxsize=None)
def _make_sc_kernels():
    mesh = plsc.VectorSubcoreMesh(
        core_axis_name="c", subcore_axis_name="s",
        num_cores=NC, num_subcores=NS)
    agg = pl.kernel(
        _sc_agg_body,
        out_type=(jax.ShapeDtypeStruct((NP, D), _f32),
                  jax.ShapeDtypeStruct((NP, D), _f32)),
        mesh=mesh,
        scratch_types=(
            pltpu.VMEM((NBUF, K), jnp.int32),
            pltpu.VMEM((NBUF, K), jnp.int32),
            pltpu.VMEM((K, D), _f32),
            pltpu.VMEM((K, D), _f32),
            pltpu.VMEM((K, D), _f32),
            pltpu.VMEM_SHARED((NP, D), _f32),
            pltpu.SemaphoreType.DMA,
            pltpu.SemaphoreType.DMA,
            pltpu.SemaphoreType.DMA,
        ))
    deg = pl.kernel(
        _sc_deg_body,
        out_type=(jax.ShapeDtypeStruct((NP, D), _f32),
                  jax.ShapeDtypeStruct((NP, D), _f32)),
        mesh=mesh,
        scratch_types=(
            pltpu.VMEM((1, K), jnp.int32),
            pltpu.VMEM((K, D), _f32),
            pltpu.VMEM((K, D), _f32),
            pltpu.VMEM_SHARED((NP, D), _f32),
        ))
    return agg, deg


def _lrelu(z):
    return jnp.where(z > 0, z, 0.01 * z)


def _dotT(a, w):
    return lax.dot_general(a, w, (((1,), (1,)), ((), ())),
                           preferred_element_type=_f32)


def _tc_sage_body(a0, a1, d0, d1, h, wl, bl, wr, z, ssum, ssq):
    deg = jnp.clip(d0[...][:, :1] + d1[...][:, :1], 1.0, None)
    m = (a0[...] + a1[...]) / deg
    zb = _lrelu(_dotT(m, wl[...]) + bl[0:1, :] + _dotT(h[...], wr[...]))
    z[...] = zb
    sb = jnp.broadcast_to(jnp.sum(zb, 0, keepdims=True), (8, D))
    qb = jnp.broadcast_to(jnp.sum(zb * zb, 0, keepdims=True), (8, D))

    @pl.when(pl.program_id(0) == 0)
    def _():
        ssum[...] = sb
        ssq[...] = qb

    @pl.when(pl.program_id(0) != 0)
    def _():
        ssum[...] += sb
        ssq[...] += qb


def _tc_bn_body(z, ssum, ssq, g, be, out):
    mu = ssum[0:1, :] * (1.0 / N)
    var = ssq[0:1, :] * (1.0 / N) - mu * mu
    inv = g[0:1, :] * lax.rsqrt(var + 1e-5)
    out[...] = (z[...] - mu) * inv + be[0:1, :]


def _tc_final_body(a0, a1, d0, d1, h, wl, bl, wr, wfc, bfc, out):
    deg = jnp.clip(d0[...][:, :1] + d1[...][:, :1], 1.0, None)
    m = (a0[...] + a1[...]) / deg
    zb = _lrelu(_dotT(m, wl[...]) + bl[0:1, :] + _dotT(h[...], wr[...]))
    out[...] = _dotT(zb, wfc[...]) + bfc[0:1, :]


_B = 1000
_GRID = N // _B

_spec_rows = pl.BlockSpec((_B, D), lambda i: (i, 0))
_spec_deg = pl.BlockSpec((_B, D), lambda i: (i, 0))
_spec_w = pl.BlockSpec((D, D), lambda i: (0, 0))
_spec_b = pl.BlockSpec((8, D), lambda i: (0, 0))

_tc_sage = pl.pallas_call(
    _tc_sage_body,
    grid=(_GRID,),
    in_specs=[_spec_rows, _spec_rows, _spec_deg, _spec_deg, _spec_rows,
              _spec_w, _spec_b, _spec_w],
    out_specs=[_spec_rows, _spec_b, _spec_b],
    out_shape=[jax.ShapeDtypeStruct((N, D), _f32),
               jax.ShapeDtypeStruct((8, D), _f32),
               jax.ShapeDtypeStruct((8, D), _f32)],
)

_tc_bn = pl.pallas_call(
    _tc_bn_body,
    grid=(_GRID,),
    in_specs=[_spec_rows, _spec_b, _spec_b, _spec_b, _spec_b],
    out_specs=_spec_rows,
    out_shape=jax.ShapeDtypeStruct((N, D), _f32),
)

_tc_final = pl.pallas_call(
    _tc_final_body,
    grid=(_GRID,),
    in_specs=[_spec_rows, _spec_rows, _spec_deg, _spec_deg, _spec_rows,
              _spec_w, _spec_b, _spec_w, _spec_w, _spec_b],
    out_specs=_spec_rows,
    out_shape=jax.ShapeDtypeStruct((N, D), _f32),
)


def _b8(v):
    return jnp.broadcast_to(v.reshape(1, D), (8, D))


def kernel(x, edge_index, Wl1, bl1, Wr1, Wl2, bl2, Wr2, Wl3, bl3, Wr3,
           g1, be1, g2, be2, Wfc, bfc):
    src = edge_index[0].astype(jnp.int32)
    dst = edge_index[1].astype(jnp.int32)
    pad = EPAD - E
    srcf = jnp.concatenate([src, jnp.zeros((pad,), jnp.int32)])
    pad_dst = PAD_ROW + jnp.arange(pad, dtype=jnp.int32) % (NP - N)
    dstf = jnp.concatenate([dst, pad_dst])

    _sc_agg, _sc_deg = _make_sc_kernels()

    d0, d1 = _sc_deg(dstf)
    d0, d1 = d0[:N], d1[:N]

    a0, a1 = _sc_agg(x, srcf, dstf)
    z1, s1, q1 = _tc_sage(a0[:N], a1[:N], d0, d1, x, Wl1, _b8(bl1), Wr1)
    h1 = _tc_bn(z1, s1, q1, _b8(g1), _b8(be1))

    a0, a1 = _sc_agg(h1, srcf, dstf)
    z2, s2, q2 = _tc_sage(a0[:N], a1[:N], d0, d1, h1, Wl2, _b8(bl2), Wr2)
    h2 = _tc_bn(z2, s2, q2, _b8(g2), _b8(be2))

    a0, a1 = _sc_agg(h2, srcf, dstf)
    out = _tc_final(a0[:N], a1[:N], d0, d1, h2, Wl3, _b8(bl3), Wr3,
                    Wfc, _b8(bfc))
    return out


# speedup vs baseline: 1.0567x; 1.0567x over previous
import functools

import jax
import jax.numpy as jnp
from jax import lax
from jax.experimental import pallas as pl
from jax.experimental.pallas import tpu as pltpu
from jax.experimental.pallas import tpu_sc as plsc

N = 10000
E = 320000
D = 128

NC, NS = 2, 16
NW = NC * NS
K = 128
NCHUNK = 80
EPW = K * NCHUNK
EPAD = NW * EPW
NP = 10112
RPS = NP // NS
PAD_ROW = N
ZROWS = 64

_f32 = jnp.float32


NBUF = 3
NG = -(-NCHUNK // NBUF)
KA, KB = 122, 38


def _zero_accum(buf, accum, s):
    z16 = jnp.zeros((16,), _f32)

    def _zb(i, _):
        for j in range(8):
            buf[i, pl.ds(j * 16, 16)] = z16
        return 0
    lax.fori_loop(0, K, _zb, 0)

    def _za(i, _):
        pltpu.sync_copy(buf, accum.at[pl.ds(s * RPS + i * K, K)])
        return 0
    lax.fori_loop(0, RPS // K, _za, 0)
    pltpu.sync_copy(buf, accum.at[pl.ds(s * RPS + RPS - K, K)])


def _sc_agg_body(h_hbm, srcf, dstf, agg0, agg1,
                 sidx, didx, rows0, rows1, rows2, accum, sem0, sem1, sem2):
    c = lax.axis_index("c")
    s = lax.axis_index("s")
    nchunk = jnp.where(c == 0, KA, KB)
    base = jnp.where(c == 0, s * (KA * K), (NS * KA + s * KB) * K)
    rows = [rows0, rows1, rows2]
    sems = [sem0, sem1, sem2]

    _zero_accum(rows0, accum, s)
    plsc.subcore_barrier()

    def _ld(slot, j):
        off = pl.multiple_of(base + j * K, K)
        pltpu.sync_copy(srcf.at[pl.ds(off, K)], sidx.at[slot])
        pltpu.sync_copy(dstf.at[pl.ds(off, K)], didx.at[slot])

    def _gather(slot):
        pltpu.async_copy(h_hbm.at[sidx.at[slot]], rows[slot], sems[slot])

    def _gwait(slot):
        pltpu.make_async_copy(
            h_hbm.at[sidx.at[slot]], rows[slot], sems[slot]).wait()

    _ld(0, 0)
    _gather(0)
    _ld(1, 1)
    _gather(1)

    def _group(p, _):
        for b in range(NBUF):
            q = NBUF * p + b
            qn = q + 2
            nb = (b + 2) % NBUF

            @pl.when(qn < nchunk)
            def _():
                _ld(nb, qn)

            @pl.when(q < nchunk)
            def _():
                _gwait(b)

                @pl.when(qn < nchunk)
                def _():
                    _gather(nb)
                pltpu.sync_copy(rows[b], accum.at[didx.at[b]], add=True)
        return 0
    lax.fori_loop(0, (nchunk + NBUF - 1) // NBUF, _group, 0)

    plsc.subcore_barrier()

    @pl.when(c == 0)
    def _():
        pltpu.sync_copy(accum.at[pl.ds(s * RPS, RPS)],
                        agg0.at[pl.ds(s * RPS, RPS)])

    @pl.when(c == 1)
    def _():
        pltpu.sync_copy(accum.at[pl.ds(s * RPS, RPS)],
                        agg1.at[pl.ds(s * RPS, RPS)])


def _sc_deg_body(dstf, deg0, deg1, didx, ones, zbuf, accum):
    c = lax.axis_index("c")
    s = lax.axis_index("s")
    wid = s * NC + c
    base = wid * EPW

    o16 = jnp.ones((16,), _f32)

    def _fill(i, _):
        for j in range(8):
            ones[i, pl.ds(j * 16, 16)] = o16
        return 0
    lax.fori_loop(0, K, _fill, 0)

    _zero_accum(zbuf, accum, s)
    plsc.subcore_barrier()

    def _chunk(j, _):
        off = pl.multiple_of(base + j * K, K)
        pltpu.sync_copy(dstf.at[pl.ds(off, K)], didx.at[0])
        pltpu.sync_copy(ones, accum.at[didx.at[0]], add=True)
        return 0
    lax.fori_loop(0, NCHUNK, _chunk, 0)

    plsc.subcore_barrier()

    @pl.when(c == 0)
    def _():
        pltpu.sync_copy(accum.at[pl.ds(s * RPS, RPS)],
                        deg0.at[pl.ds(s * RPS, RPS)])

    @pl.when(c == 1)
    def _():
        pltpu.sync_copy(accum.at[pl.ds(s * RPS, RPS)],
                        deg1.at[pl.ds(s * RPS, RPS)])


@functools.lru_cache(maxsize=None)
def _make_sc_kernels():
    mesh = plsc.VectorSubcoreMesh(
        core_axis_name="c", subcore_axis_name="s",
        num_cores=NC, num_subcores=NS)
    agg = pl.kernel(
        _sc_agg_body,
        out_type=(jax.ShapeDtypeStruct((NP, D), _f32),
                  jax.ShapeDtypeStruct((NP, D), _f32)),
        mesh=mesh,
        scratch_types=(
            pltpu.VMEM((NBUF, K), jnp.int32),
            pltpu.VMEM((NBUF, K), jnp.int32),
            pltpu.VMEM((K, D), _f32),
            pltpu.VMEM((K, D), _f32),
            pltpu.VMEM((K, D), _f32),
            pltpu.VMEM_SHARED((NP, D), _f32),
            pltpu.SemaphoreType.DMA,
            pltpu.SemaphoreType.DMA,
            pltpu.SemaphoreType.DMA,
        ))
    deg = pl.kernel(
        _sc_deg_body,
        out_type=(jax.ShapeDtypeStruct((NP, D), _f32),
                  jax.ShapeDtypeStruct((NP, D), _f32)),
        mesh=mesh,
        scratch_types=(
            pltpu.VMEM((1, K), jnp.int32),
            pltpu.VMEM((K, D), _f32),
            pltpu.VMEM((K, D), _f32),
            pltpu.VMEM_SHARED((NP, D), _f32),
        ))
    return agg, deg


def _lrelu(z):
    return jnp.where(z > 0, z, 0.01 * z)


def _dotT(a, w):
    return lax.dot_general(a, w, (((1,), (1,)), ((), ())),
                           preferred_element_type=_f32)


def _tc_sage_body(a0, a1, d0, d1, h, wl, bl, wr, z, ssum, ssq):
    deg = jnp.clip(d0[...][:, :1] + d1[...][:, :1], 1.0, None)
    m = (a0[...] + a1[...]) / deg
    zb = _lrelu(_dotT(m, wl[...]) + bl[0:1, :] + _dotT(h[...], wr[...]))
    z[...] = zb
    sb = jnp.broadcast_to(jnp.sum(zb, 0, keepdims=True), (8, D))
    qb = jnp.broadcast_to(jnp.sum(zb * zb, 0, keepdims=True), (8, D))

    @pl.when(pl.program_id(0) == 0)
    def _():
        ssum[...] = sb
        ssq[...] = qb

    @pl.when(pl.program_id(0) != 0)
    def _():
        ssum[...] += sb
        ssq[...] += qb


def _tc_bn_body(z, ssum, ssq, g, be, out):
    mu = ssum[0:1, :] * (1.0 / N)
    var = ssq[0:1, :] * (1.0 / N) - mu * mu
    inv = g[0:1, :] * lax.rsqrt(var + 1e-5)
    out[...] = (z[...] - mu) * inv + be[0:1, :]


def _tc_final_body(a0, a1, d0, d1, h, wl, bl, wr, wfc, bfc, out):
    deg = jnp.clip(d0[...][:, :1] + d1[...][:, :1], 1.0, None)
    m = (a0[...] + a1[...]) / deg
    zb = _lrelu(_dotT(m, wl[...]) + bl[0:1, :] + _dotT(h[...], wr[...]))
    out[...] = _dotT(zb, wfc[...]) + bfc[0:1, :]


_B = 1000
_GRID = N // _B

_spec_rows = pl.BlockSpec((_B, D), lambda i: (i, 0))
_spec_deg = pl.BlockSpec((_B, D), lambda i: (i, 0))
_spec_w = pl.BlockSpec((D, D), lambda i: (0, 0))
_spec_b = pl.BlockSpec((8, D), lambda i: (0, 0))

_tc_sage = pl.pallas_call(
    _tc_sage_body,
    grid=(_GRID,),
    in_specs=[_spec_rows, _spec_rows, _spec_deg, _spec_deg, _spec_rows,
              _spec_w, _spec_b, _spec_w],
    out_specs=[_spec_rows, _spec_b, _spec_b],
    out_shape=[jax.ShapeDtypeStruct((N, D), _f32),
               jax.ShapeDtypeStruct((8, D), _f32),
               jax.ShapeDtypeStruct((8, D), _f32)],
)

_tc_bn = pl.pallas_call(
    _tc_bn_body,
    grid=(_GRID,),
    in_specs=[_spec_rows, _spec_b, _spec_b, _spec_b, _spec_b],
    out_specs=_spec_rows,
    out_shape=jax.ShapeDtypeStruct((N, D), _f32),
)

_tc_final = pl.pallas_call(
    _tc_final_body,
    grid=(_GRID,),
    in_specs=[_spec_rows, _spec_rows, _spec_deg, _spec_deg, _spec_rows,
              _spec_w, _spec_b, _spec_w, _spec_w, _spec_b],
    out_specs=_spec_rows,
    out_shape=jax.ShapeDtypeStruct((N, D), _f32),
)


def _b8(v):
    return jnp.broadcast_to(v.reshape(1, D), (8, D))


def kernel(x, edge_index, Wl1, bl1, Wr1, Wl2, bl2, Wr2, Wl3, bl3, Wr3,
           g1, be1, g2, be2, Wfc, bfc):
    src = edge_index[0].astype(jnp.int32)
    dst = edge_index[1].astype(jnp.int32)
    pad = EPAD - E
    srcf = jnp.concatenate([src, jnp.zeros((pad,), jnp.int32)])
    pad_dst = PAD_ROW + jnp.arange(pad, dtype=jnp.int32) % (NP - N)
    dstf = jnp.concatenate([dst, pad_dst])

    _sc_agg, _sc_deg = _make_sc_kernels()

    d0, d1 = _sc_deg(dstf)
    d0, d1 = d0[:N], d1[:N]

    a0, a1 = _sc_agg(x, srcf, dstf)
    z1, s1, q1 = _tc_sage(a0[:N], a1[:N], d0, d1, x, Wl1, _b8(bl1), Wr1)
    h1 = _tc_bn(z1, s1, q1, _b8(g1), _b8(be1))

    a0, a1 = _sc_agg(h1, srcf, dstf)
    z2, s2, q2 = _tc_sage(a0[:N], a1[:N], d0, d1, h1, Wl2, _b8(bl2), Wr2)
    h2 = _tc_bn(z2, s2, q2, _b8(g2), _b8(be2))

    a0, a1 = _sc_agg(h2, srcf, dstf)
    out = _tc_final(a0[:N], a1[:N], d0, d1, h2, Wl3, _b8(bl3), Wr3,
                    Wfc, _b8(bfc))
    return out


# speedup vs baseline: 1.1132x; 1.0534x over previous
import functools

import jax
import jax.numpy as jnp
from jax import lax
from jax.experimental import pallas as pl
from jax.experimental.pallas import tpu as pltpu
from jax.experimental.pallas import tpu_sc as plsc

N = 10000
E = 320000
D = 128

NC, NS = 2, 16
NW = NC * NS
K = 128
NCHUNK = 80
EPW = K * NCHUNK
EPAD = NW * EPW
NP = 10112
RPS = NP // NS
PAD_ROW = N
ZROWS = 64

_f32 = jnp.float32


NBUF = 3
NG = -(-NCHUNK // NBUF)
KA, KB = 138, 22


def _zero_accum(buf, accum, s):
    z16 = jnp.zeros((16,), _f32)

    def _zb(i, _):
        for j in range(8):
            buf[i, pl.ds(j * 16, 16)] = z16
        return 0
    lax.fori_loop(0, K, _zb, 0)

    def _za(i, _):
        pltpu.sync_copy(buf, accum.at[pl.ds(s * RPS + i * K, K)])
        return 0
    lax.fori_loop(0, RPS // K, _za, 0)
    pltpu.sync_copy(buf, accum.at[pl.ds(s * RPS + RPS - K, K)])


def _sc_agg_body(h_hbm, srcf, dstf, agg0, agg1,
                 sidx, didx, rows0, rows1, rows2, accum, sem0, sem1, sem2):
    c = lax.axis_index("c")
    s = lax.axis_index("s")
    nchunk = jnp.where(c == 0, KA, KB)
    base = jnp.where(c == 0, s * (KA * K), (NS * KA + s * KB) * K)
    rows = [rows0, rows1, rows2]
    sems = [sem0, sem1, sem2]

    _zero_accum(rows0, accum, s)
    plsc.subcore_barrier()

    def _ld(slot, j):
        off = pl.multiple_of(base + j * K, K)
        pltpu.sync_copy(srcf.at[pl.ds(off, K)], sidx.at[slot])
        pltpu.sync_copy(dstf.at[pl.ds(off, K)], didx.at[slot])

    def _gather(slot):
        pltpu.async_copy(h_hbm.at[sidx.at[slot]], rows[slot], sems[slot])

    def _gwait(slot):
        pltpu.make_async_copy(
            h_hbm.at[sidx.at[slot]], rows[slot], sems[slot]).wait()

    _ld(0, 0)
    _gather(0)
    _ld(1, 1)
    _gather(1)

    def _group(p, _):
        for b in range(NBUF):
            q = NBUF * p + b
            qn = q + 2
            nb = (b + 2) % NBUF

            @pl.when(qn < nchunk)
            def _():
                _ld(nb, qn)

            @pl.when(q < nchunk)
            def _():
                _gwait(b)

                @pl.when(qn < nchunk)
                def _():
                    _gather(nb)
                pltpu.sync_copy(rows[b], accum.at[didx.at[b]], add=True)
        return 0
    lax.fori_loop(0, (nchunk + NBUF - 1) // NBUF, _group, 0)

    plsc.subcore_barrier()

    @pl.when(c == 0)
    def _():
        pltpu.sync_copy(accum.at[pl.ds(s * RPS, RPS)],
                        agg0.at[pl.ds(s * RPS, RPS)])

    @pl.when(c == 1)
    def _():
        pltpu.sync_copy(accum.at[pl.ds(s * RPS, RPS)],
                        agg1.at[pl.ds(s * RPS, RPS)])


def _sc_deg_body(dstf, deg0, deg1, didx, ones, zbuf, accum):
    c = lax.axis_index("c")
    s = lax.axis_index("s")
    wid = s * NC + c
    base = wid * EPW

    o16 = jnp.ones((16,), _f32)

    def _fill(i, _):
        for j in range(8):
            ones[i, pl.ds(j * 16, 16)] = o16
        return 0
    lax.fori_loop(0, K, _fill, 0)

    _zero_accum(zbuf, accum, s)
    plsc.subcore_barrier()

    def _chunk(j, _):
        off = pl.multiple_of(base + j * K, K)
        pltpu.sync_copy(dstf.at[pl.ds(off, K)], didx.at[0])
        pltpu.sync_copy(ones, accum.at[didx.at[0]], add=True)
        return 0
    lax.fori_loop(0, NCHUNK, _chunk, 0)

    plsc.subcore_barrier()

    @pl.when(c == 0)
    def _():
        pltpu.sync_copy(accum.at[pl.ds(s * RPS, RPS)],
                        deg0.at[pl.ds(s * RPS, RPS)])

    @pl.when(c == 1)
    def _():
        pltpu.sync_copy(accum.at[pl.ds(s * RPS, RPS)],
                        deg1.at[pl.ds(s * RPS, RPS)])


@functools.lru_cache(maxsize=None)
def _make_sc_kernels():
    mesh = plsc.VectorSubcoreMesh(
        core_axis_name="c", subcore_axis_name="s",
        num_cores=NC, num_subcores=NS)
    agg = pl.kernel(
        _sc_agg_body,
        out_type=(jax.ShapeDtypeStruct((NP, D), _f32),
                  jax.ShapeDtypeStruct((NP, D), _f32)),
        mesh=mesh,
        scratch_types=(
            pltpu.VMEM((NBUF, K), jnp.int32),
            pltpu.VMEM((NBUF, K), jnp.int32),
            pltpu.VMEM((K, D), _f32),
            pltpu.VMEM((K, D), _f32),
            pltpu.VMEM((K, D), _f32),
            pltpu.VMEM_SHARED((NP, D), _f32),
            pltpu.SemaphoreType.DMA,
            pltpu.SemaphoreType.DMA,
            pltpu.SemaphoreType.DMA,
        ))
    deg = pl.kernel(
        _sc_deg_body,
        out_type=(jax.ShapeDtypeStruct((NP, D), _f32),
                  jax.ShapeDtypeStruct((NP, D), _f32)),
        mesh=mesh,
        scratch_types=(
            pltpu.VMEM((1, K), jnp.int32),
            pltpu.VMEM((K, D), _f32),
            pltpu.VMEM((K, D), _f32),
            pltpu.VMEM_SHARED((NP, D), _f32),
        ))
    return agg, deg


def _lrelu(z):
    return jnp.where(z > 0, z, 0.01 * z)


def _dotT(a, w):
    return lax.dot_general(a, w, (((1,), (1,)), ((), ())),
                           preferred_element_type=_f32)


def _tc_sage_body(a0, a1, d0, d1, h, wl, bl, wr, z, ssum, ssq):
    deg = jnp.clip(d0[...][:, :1] + d1[...][:, :1], 1.0, None)
    m = (a0[...] + a1[...]) / deg
    zb = _lrelu(_dotT(m, wl[...]) + bl[0:1, :] + _dotT(h[...], wr[...]))
    z[...] = zb
    sb = jnp.broadcast_to(jnp.sum(zb, 0, keepdims=True), (8, D))
    qb = jnp.broadcast_to(jnp.sum(zb * zb, 0, keepdims=True), (8, D))

    @pl.when(pl.program_id(0) == 0)
    def _():
        ssum[...] = sb
        ssq[...] = qb

    @pl.when(pl.program_id(0) != 0)
    def _():
        ssum[...] += sb
        ssq[...] += qb


def _tc_bn_body(z, ssum, ssq, g, be, out):
    mu = ssum[0:1, :] * (1.0 / N)
    var = ssq[0:1, :] * (1.0 / N) - mu * mu
    inv = g[0:1, :] * lax.rsqrt(var + 1e-5)
    out[...] = (z[...] - mu) * inv + be[0:1, :]


def _tc_final_body(a0, a1, d0, d1, h, wl, bl, wr, wfc, bfc, out):
    deg = jnp.clip(d0[...][:, :1] + d1[...][:, :1], 1.0, None)
    m = (a0[...] + a1[...]) / deg
    zb = _lrelu(_dotT(m, wl[...]) + bl[0:1, :] + _dotT(h[...], wr[...]))
    out[...] = _dotT(zb, wfc[...]) + bfc[0:1, :]


_B = 1000
_GRID = N // _B

_spec_rows = pl.BlockSpec((_B, D), lambda i: (i, 0))
_spec_deg = pl.BlockSpec((_B, D), lambda i: (i, 0))
_spec_w = pl.BlockSpec((D, D), lambda i: (0, 0))
_spec_b = pl.BlockSpec((8, D), lambda i: (0, 0))

_tc_sage = pl.pallas_call(
    _tc_sage_body,
    grid=(_GRID,),
    in_specs=[_spec_rows, _spec_rows, _spec_deg, _spec_deg, _spec_rows,
              _spec_w, _spec_b, _spec_w],
    out_specs=[_spec_rows, _spec_b, _spec_b],
    out_shape=[jax.ShapeDtypeStruct((N, D), _f32),
               jax.ShapeDtypeStruct((8, D), _f32),
               jax.ShapeDtypeStruct((8, D), _f32)],
)

_tc_bn = pl.pallas_call(
    _tc_bn_body,
    grid=(_GRID,),
    in_specs=[_spec_rows, _spec_b, _spec_b, _spec_b, _spec_b],
    out_specs=_spec_rows,
    out_shape=jax.ShapeDtypeStruct((N, D), _f32),
)

_tc_final = pl.pallas_call(
    _tc_final_body,
    grid=(_GRID,),
    in_specs=[_spec_rows, _spec_rows, _spec_deg, _spec_deg, _spec_rows,
              _spec_w, _spec_b, _spec_w, _spec_w, _spec_b],
    out_specs=_spec_rows,
    out_shape=jax.ShapeDtypeStruct((N, D), _f32),
)


def _b8(v):
    return jnp.broadcast_to(v.reshape(1, D), (8, D))


def kernel(x, edge_index, Wl1, bl1, Wr1, Wl2, bl2, Wr2, Wl3, bl3, Wr3,
           g1, be1, g2, be2, Wfc, bfc):
    src = edge_index[0].astype(jnp.int32)
    dst = edge_index[1].astype(jnp.int32)
    pad = EPAD - E
    srcf = jnp.concatenate([src, jnp.zeros((pad,), jnp.int32)])
    pad_dst = PAD_ROW + jnp.arange(pad, dtype=jnp.int32) % (NP - N)
    dstf = jnp.concatenate([dst, pad_dst])

    _sc_agg, _sc_deg = _make_sc_kernels()

    d0, d1 = _sc_deg(dstf)
    d0, d1 = d0[:N], d1[:N]

    a0, a1 = _sc_agg(x, srcf, dstf)
    z1, s1, q1 = _tc_sage(a0[:N], a1[:N], d0, d1, x, Wl1, _b8(bl1), Wr1)
    h1 = _tc_bn(z1, s1, q1, _b8(g1), _b8(be1))

    a0, a1 = _sc_agg(h1, srcf, dstf)
    z2, s2, q2 = _tc_sage(a0[:N], a1[:N], d0, d1, h1, Wl2, _b8(bl2), Wr2)
    h2 = _tc_bn(z2, s2, q2, _b8(g2), _b8(be2))

    a0, a1 = _sc_agg(h2, srcf, dstf)
    out = _tc_final(a0[:N], a1[:N], d0, d1, h2, Wl3, _b8(bl3), Wr3,
                    Wfc, _b8(bfc))
    return out
